# baseline (speedup 1.0000x reference)
import functools

import jax
import jax.numpy as jnp
from jax import lax
from jax.experimental import pallas as pl
from jax.experimental.pallas import tpu as pltpu
from jax.experimental.pallas import tpu_sc as plsc


def kernel(input_ids, weight):
    seq_len = input_ids.shape[1]
    dim = weight.shape[1]

    info = plsc.get_sparse_core_info()
    num_cores, num_subcores = info.num_cores, info.num_subcores
    num_workers = num_cores * num_subcores
    rows_per_worker = seq_len // num_workers

    mesh = plsc.VectorSubcoreMesh(core_axis_name="c", subcore_axis_name="s")

    n_chunks = 8
    rows_per_chunk = rows_per_worker // n_chunks

    @functools.partial(
        pl.kernel,
        mesh=mesh,
        out_type=jax.ShapeDtypeStruct((seq_len, dim), weight.dtype),
        scratch_types=[pltpu.SemaphoreType.DMA],
    )
    def positional_rows_copy(table_hbm, out_hbm, sem):
        wid = lax.axis_index("s") * num_cores + lax.axis_index("c")
        base = wid * rows_per_worker
        copies = [
            pltpu.make_async_copy(
                table_hbm.at[pl.ds(base + i * rows_per_chunk, rows_per_chunk)],
                out_hbm.at[pl.ds(base + i * rows_per_chunk, rows_per_chunk)],
                sem,
            )
            for i in range(n_chunks)
        ]
        for c in copies:
            c.start()
        for c in copies:
            c.wait()

    return positional_rows_copy(weight)


# speedup vs baseline: 16.6310x; 16.6310x over previous
import functools

import jax
import jax.numpy as jnp
from jax import lax
from jax.experimental import pallas as pl
from jax.experimental.pallas import tpu as pltpu
from jax.experimental.pallas import tpu_sc as plsc


def kernel(input_ids, weight):
    seq_len = input_ids.shape[1]
    dim = weight.shape[1]

    info = plsc.get_sparse_core_info()
    num_cores, num_subcores = info.num_cores, info.num_subcores
    num_workers = num_cores * num_subcores
    rows_per_worker = seq_len // num_workers

    mesh = plsc.VectorSubcoreMesh(core_axis_name="c", subcore_axis_name="s")

    rows_per_chunk = 32
    n_chunks = rows_per_worker // rows_per_chunk

    @functools.partial(
        pl.kernel,
        mesh=mesh,
        out_type=jax.ShapeDtypeStruct((seq_len, dim), weight.dtype),
        scratch_types=[
            pltpu.VMEM((rows_per_chunk, dim), jnp.float32),
            pltpu.VMEM((rows_per_chunk, dim), jnp.float32),
            pltpu.SemaphoreType.DMA,
            pltpu.SemaphoreType.DMA,
            pltpu.SemaphoreType.DMA,
            pltpu.SemaphoreType.DMA,
        ],
    )
    def positional_rows_copy(table_hbm, out_hbm, buf0, buf1, ls0, ls1, ss0, ss1):
        wid = lax.axis_index("s") * num_cores + lax.axis_index("c")
        base = wid * rows_per_worker
        bufs = (buf0, buf1)
        lsems = (ls0, ls1)
        ssems = (ss0, ss1)
        loads = [
            pltpu.make_async_copy(
                table_hbm.at[pl.ds(base + i * rows_per_chunk, rows_per_chunk)],
                bufs[i % 2],
                lsems[i % 2],
            )
            for i in range(n_chunks)
        ]
        stores = [
            pltpu.make_async_copy(
                bufs[i % 2],
                out_hbm.at[pl.ds(base + i * rows_per_chunk, rows_per_chunk)],
                ssems[i % 2],
            )
            for i in range(n_chunks)
        ]
        loads[0].start()
        for i in range(n_chunks):
            if i + 1 < n_chunks:
                if i - 1 >= 0:
                    stores[i - 1].wait()
                loads[i + 1].start()
            loads[i].wait()
            stores[i].start()
        if n_chunks >= 2:
            stores[n_chunks - 2].wait()
        stores[n_chunks - 1].wait()

    return positional_rows_copy(weight)


# speedup vs baseline: 16.8044x; 1.0104x over previous
import functools

import jax
import jax.numpy as jnp
from jax import lax
from jax.experimental import pallas as pl
from jax.experimental.pallas import tpu as pltpu
from jax.experimental.pallas import tpu_sc as plsc


def kernel(input_ids, weight):
    seq_len = input_ids.shape[1]
    dim = weight.shape[1]

    info = plsc.get_sparse_core_info()
    num_cores, num_subcores = info.num_cores, info.num_subcores
    num_workers = num_cores * num_subcores
    rows_per_worker = seq_len // num_workers

    mesh = plsc.VectorSubcoreMesh(core_axis_name="c", subcore_axis_name="s")

    chunk_rows = (8, 24, 32, 32, 32)
    assert sum(chunk_rows) == rows_per_worker
    chunk_buf = (0, 1, 2, 3, 2)
    buf_rows = (8, 24, 32, 32)

    @functools.partial(
        pl.kernel,
        mesh=mesh,
        out_type=jax.ShapeDtypeStruct((seq_len, dim), weight.dtype),
        scratch_types=(
            [pltpu.VMEM((r, dim), jnp.float32) for r in buf_rows]
            + [pltpu.SemaphoreType.DMA] * 8
        ),
    )
    def positional_rows_copy(table_hbm, out_hbm, *scratch):
        bufs = scratch[:4]
        lsems = scratch[4:8]
        ssems = scratch[8:12]
        wid = lax.axis_index("s") * num_cores + lax.axis_index("c")
        base = wid * rows_per_worker
        offs = []
        o = 0
        for r in chunk_rows:
            offs.append(o)
            o += r
        loads = [
            pltpu.make_async_copy(
                table_hbm.at[pl.ds(base + offs[i], chunk_rows[i])],
                bufs[chunk_buf[i]],
                lsems[chunk_buf[i]],
            )
            for i in range(len(chunk_rows))
        ]
        stores = [
            pltpu.make_async_copy(
                bufs[chunk_buf[i]],
                out_hbm.at[pl.ds(base + offs[i], chunk_rows[i])],
                ssems[chunk_buf[i]],
            )
            for i in range(len(chunk_rows))
        ]
        loads[0].start()
        loads[1].start()
        loads[2].start()
        loads[3].start()
        loads[0].wait()
        stores[0].start()
        loads[1].wait()
        stores[1].start()
        loads[2].wait()
        stores[2].start()
        loads[3].wait()
        stores[3].start()
        stores[2].wait()
        loads[4].start()
        loads[4].wait()
        stores[4].start()
        stores[0].wait()
        stores[1].wait()
        stores[3].wait()
        stores[4].wait()

    return positional_rows_copy(weight)


# speedup vs baseline: 17.4597x; 1.0390x over previous
import functools

import jax
import jax.numpy as jnp
from jax import lax
from jax.experimental import pallas as pl
from jax.experimental.pallas import tpu as pltpu
from jax.experimental.pallas import tpu_sc as plsc


def kernel(input_ids, weight):
    seq_len = input_ids.shape[1]
    dim = weight.shape[1]

    info = plsc.get_sparse_core_info()
    num_cores, num_subcores = info.num_cores, info.num_subcores
    num_workers = num_cores * num_subcores
    rows_per_worker = seq_len // num_workers

    mesh = plsc.VectorSubcoreMesh(core_axis_name="c", subcore_axis_name="s")

    chunk_rows = (8, 16, 24, 32, 40, 8)
    chunk_buf = (0, 1, 2, 3, 4, 0)
    buf_rows = (8, 16, 24, 32, 40)
    assert sum(chunk_rows) == rows_per_worker
    n_chunks = len(chunk_rows)
    n_bufs = len(buf_rows)

    @functools.partial(
        pl.kernel,
        mesh=mesh,
        out_type=jax.ShapeDtypeStruct((seq_len, dim), weight.dtype),
        scratch_types=(
            [pltpu.VMEM((r, dim), jnp.float32) for r in buf_rows]
            + [pltpu.SemaphoreType.DMA] * (2 * n_bufs)
        ),
    )
    def positional_rows_copy(table_hbm, out_hbm, *scratch):
        bufs = scratch[:n_bufs]
        lsems = scratch[n_bufs : 2 * n_bufs]
        ssems = scratch[2 * n_bufs :]
        wid = lax.axis_index("s") * num_cores + lax.axis_index("c")
        base = wid * rows_per_worker
        offs = []
        o = 0
        for r in chunk_rows:
            offs.append(o)
            o += r
        loads = [
            pltpu.make_async_copy(
                table_hbm.at[pl.ds(base + offs[i], chunk_rows[i])],
                bufs[chunk_buf[i]],
                lsems[chunk_buf[i]],
            )
            for i in range(n_chunks)
        ]
        stores = [
            pltpu.make_async_copy(
                bufs[chunk_buf[i]],
                out_hbm.at[pl.ds(base + offs[i], chunk_rows[i])],
                ssems[chunk_buf[i]],
            )
            for i in range(n_chunks)
        ]
        for i in range(n_bufs):
            loads[i].start()
        loads[0].wait()
        stores[0].start()
        loads[1].wait()
        stores[1].start()
        stores[0].wait()
        loads[5].start()
        loads[2].wait()
        stores[2].start()
        loads[3].wait()
        stores[3].start()
        loads[4].wait()
        stores[4].start()
        loads[5].wait()
        stores[5].start()
        stores[1].wait()
        stores[2].wait()
        stores[3].wait()
        stores[4].wait()
        stores[5].wait()

    return positional_rows_copy(weight)
